# baseline (speedup 1.0000x reference)
import functools

import jax
import jax.numpy as jnp
from jax import lax
from jax.experimental import pallas as pl
from jax.experimental.pallas import tpu as pltpu
from jax.experimental.pallas import tpu_sc as plsc

VOCAB = 32000
D = 1024
E = 8
C = 640
F = 2048
T = 2048
NH = 2
EC = E * C

NC, NS, L = 2, 16, 16
NW = NC * NS
ROWS_PW = T // NW
SLOTS_PW = EC // NW
GCH = 80

BF = jnp.bfloat16
F32 = jnp.float32
I32 = jnp.int32

_mesh = plsc.VectorSubcoreMesh(core_axis_name="c", subcore_axis_name="s")
_sc_params = pltpu.CompilerParams(needs_layout_passes=False)


def _wid():
    return lax.axis_index("s") * NC + lax.axis_index("c")


def _rnd_bf16(v):
    x = plsc.bitcast(v, I32)
    r = (x + 32767 + ((x >> 16) & 1)) & jnp.int32(-65536)
    return plsc.bitcast(r, F32)


def _embed_body(tab, ids, out, idx_v, rows_v, sem):
    base = _wid() * ROWS_PW
    pltpu.sync_copy(ids.at[pl.ds(base, ROWS_PW)], idx_v)
    pltpu.async_copy(tab.at[idx_v], rows_v, sem).wait()
    pltpu.sync_copy(rows_v, out.at[pl.ds(base, ROWS_PW)])


_embed_gather = pl.kernel(
    _embed_body,
    compiler_params=_sc_params,
    out_type=jax.ShapeDtypeStruct((T, D), F32),
    mesh=_mesh,
    scratch_types=[
        pltpu.VMEM((ROWS_PW,), I32),
        pltpu.VMEM((ROWS_PW, D), F32),
        pltpu.SemaphoreType.DMA,
    ],
)


def _router_body(h_ref, rw_ref, rb_ref, d1r, d2r, c1r, c2r, w1r, w2r, rhor):
    h = h_ref[...]
    lg = lax.dot_general(h.astype(BF), rw_ref[...], (((1,), (0,)), ((), ())),
                         preferred_element_type=F32)
    lane = lax.broadcasted_iota(I32, (T, 128), 1)
    real = lane < E
    lg = jnp.where(real, lg + rb_ref[...], -1e30)
    m = jnp.max(lg, axis=1, keepdims=True)
    ex = jnp.where(real, jnp.exp(lg - m), 0.0)
    p = ex / jnp.sum(ex, axis=1, keepdims=True)
    m1 = jnp.max(p, axis=1, keepdims=True)
    i1 = jnp.min(jnp.where(p == m1, lane, 128), axis=1, keepdims=True)
    s1 = lane == i1
    p_x = jnp.where(s1, -1.0, p)
    m2 = jnp.max(p_x, axis=1, keepdims=True)
    i2 = jnp.min(jnp.where(p_x == m2, lane, 128), axis=1, keepdims=True)
    s2 = lane == i2
    maskf = jnp.where(s1 | s2, 1.0, 0.0)
    cs = maskf
    sh = 1
    while sh < T:
        cs = cs + jnp.concatenate([jnp.zeros((sh, 128), F32), cs[:T - sh]], axis=0)
        sh *= 2
    pos = cs - 1.0
    p1 = jnp.sum(jnp.where(s1, pos, 0.0), axis=1, keepdims=True)
    p2 = jnp.sum(jnp.where(s2, pos, 0.0), axis=1, keepdims=True)
    w1 = jnp.sum(jnp.where(s1, p, 0.0), axis=1, keepdims=True)
    w2 = jnp.sum(jnp.where(s2, p, 0.0), axis=1, keepdims=True)
    k1 = p1 < C
    k2 = p2 < C
    e1f = i1.astype(F32)
    e2f = i2.astype(F32)
    d1r[...] = jnp.where(k1, e1f * C + p1, float(EC)).astype(I32)
    d2r[...] = jnp.where(k2, e2f * C + p2, float(EC)).astype(I32)
    c1r[...] = jnp.where(k1, e1f * C + p1, 0.0).astype(I32)
    c2r[...] = jnp.where(k2, e2f * C + p2, 0.0).astype(I32)
    w1o = jnp.where(k1, w1, 0.0)
    w2o = jnp.where(k2, w2, 0.0)
    w1r[...] = w1o
    w2r[...] = w2o
    rhor[...] = w1o + w2o


_router = pl.pallas_call(
    _router_body,
    out_shape=[jax.ShapeDtypeStruct((T, 1), I32)] * 4
    + [jax.ShapeDtypeStruct((T, 1), F32)] * 3,
)


def _disp_body(h_hbm, d1_hbm, d2_hbm, out_hbm, d1_v, d2_v, s2t_v, rows_v, sem):
    pltpu.sync_copy(d1_hbm, d1_v)
    pltpu.sync_copy(d2_hbm, d2_v)

    def z_body(i, _):
        s2t_v[pl.ds(i * L, L)] = jnp.zeros((L,), I32)
        return 0

    lax.fori_loop(0, (EC + L) // L, z_body, 0)

    def sc_body(i, _):
        vals = lax.iota(I32, L) + i * L
        plsc.store_scatter(s2t_v, [d1_v[pl.ds(i * L, L)]], vals)
        plsc.store_scatter(s2t_v, [d2_v[pl.ds(i * L, L)]], vals)
        return 0

    lax.fori_loop(0, T // L, sc_body, 0)
    for ch in range(SLOTS_PW // GCH):
        base = _wid() * SLOTS_PW + ch * GCH
        pltpu.async_copy(h_hbm.at[s2t_v.at[pl.ds(base, GCH)]], rows_v, sem).wait()
        pltpu.sync_copy(rows_v, out_hbm.at[pl.ds(base, GCH)])


_dispatch = pl.kernel(
    _disp_body,
    compiler_params=_sc_params,
    out_type=jax.ShapeDtypeStruct((EC, D), F32),
    mesh=_mesh,
    scratch_types=[
        pltpu.VMEM((T,), I32),
        pltpu.VMEM((T,), I32),
        pltpu.VMEM((EC + L,), I32),
        pltpu.VMEM((GCH, D), F32),
        pltpu.SemaphoreType.DMA,
    ],
)


def _ffn_body(x_ref, w1_ref, b1_ref, w2_ref, b2_ref, o_ref):
    x = x_ref[0].astype(BF)
    h1 = lax.dot_general(x, w1_ref[0], (((1,), (0,)), ((), ())),
                         preferred_element_type=F32)
    h1 = jnp.maximum(h1 + b1_ref[0], 0.0)
    o = lax.dot_general(h1.astype(BF), w2_ref[0], (((1,), (0,)), ((), ())),
                        preferred_element_type=F32)
    o_ref[0] = o + b2_ref[0]


_ffn = pl.pallas_call(
    _ffn_body,
    grid=(E,),
    in_specs=[
        pl.BlockSpec((1, C, D), lambda e: (e, 0, 0)),
        pl.BlockSpec((1, D, F), lambda e: (e, 0, 0)),
        pl.BlockSpec((1, 1, F), lambda e: (e, 0, 0)),
        pl.BlockSpec((1, F, D), lambda e: (e, 0, 0)),
        pl.BlockSpec((1, 1, D), lambda e: (e, 0, 0)),
    ],
    out_specs=pl.BlockSpec((1, C, D), lambda e: (e, 0, 0)),
    out_shape=jax.ShapeDtypeStruct((E, C, D), F32),
)


CH_T = 16

def _comb_body(h_hbm, eo_hbm, c1_hbm, c2_hbm, w1_hbm, w2_hbm, rho_hbm, out_hbm,
               c1_v, c2_v, w1_v, w2_v, rho_v, h_v, r1_v, r2_v, o_v, sem):
    tb = _wid() * ROWS_PW
    for ch in range(ROWS_PW // CH_T):
        t0 = tb + ch * CH_T
        pltpu.sync_copy(c1_hbm.at[pl.ds(t0, CH_T)], c1_v)
        pltpu.sync_copy(c2_hbm.at[pl.ds(t0, CH_T)], c2_v)
        pltpu.sync_copy(w1_hbm.at[pl.ds(t0, CH_T)], w1_v.at[pl.ds(0, CH_T)])
        pltpu.sync_copy(w2_hbm.at[pl.ds(t0, CH_T)], w2_v.at[pl.ds(0, CH_T)])
        pltpu.sync_copy(rho_hbm.at[pl.ds(t0, CH_T)], rho_v.at[pl.ds(0, CH_T)])
        pltpu.sync_copy(h_hbm.at[pl.ds(t0, CH_T)], h_v)
        pltpu.async_copy(eo_hbm.at[c1_v], r1_v, sem).wait()
        pltpu.async_copy(eo_hbm.at[c2_v], r2_v, sem).wait()

        def tok_body(j, _):
            w1b = _rnd_bf16(jnp.full((L,), w1_v[pl.ds(j, L)][0]))
            w2b = _rnd_bf16(jnp.full((L,), w2_v[pl.ds(j, L)][0]))
            rr = rho_v[pl.ds(j, L)][0]

            def d_body(d, _):
                sl = pl.ds(d * L, L)
                hv = h_v[j, sl]
                r1 = _rnd_bf16(r1_v[j, sl])
                r2 = _rnd_bf16(r2_v[j, sl])
                o_v[j, sl] = hv + (w1b * r1 + w2b * r2) - rr * hv
                return 0

            lax.fori_loop(0, D // L, d_body, 0)
            return 0

        lax.fori_loop(0, CH_T, tok_body, 0)
        pltpu.sync_copy(o_v, out_hbm.at[pl.ds(t0, CH_T)])


_combine = pl.kernel(
    _comb_body,
    compiler_params=_sc_params,
    out_type=jax.ShapeDtypeStruct((T, D), F32),
    mesh=_mesh,
    scratch_types=[
        pltpu.VMEM((CH_T,), I32),
        pltpu.VMEM((CH_T,), I32),
        pltpu.VMEM((CH_T + L,), F32),
        pltpu.VMEM((CH_T + L,), F32),
        pltpu.VMEM((CH_T + L,), F32),
        pltpu.VMEM((CH_T, D), F32),
        pltpu.VMEM((CH_T, D), F32),
        pltpu.VMEM((CH_T, D), F32),
        pltpu.VMEM((CH_T, D), F32),
        pltpu.SemaphoreType.DMA,
    ],
)


NB = 50
NBLK = VOCAB // NB

def _final_body(h_ref, ln_ref, w_ref, o_ref, nrm_ref):
    @pl.when(pl.program_id(0) == 0)
    def _():
        h = h_ref[...]
        mean = jnp.mean(h * h, axis=1, keepdims=True)
        nrm_ref[...] = (h * lax.rsqrt(mean + 1e-6) * ln_ref[...]).astype(BF)

    o_ref[...] = lax.dot_general(nrm_ref[...], w_ref[...], (((1,), (1,)), ((), ())),
                                 preferred_element_type=F32)


_final = pl.pallas_call(
    _final_body,
    grid=(NB,),
    in_specs=[
        pl.BlockSpec((T, D), lambda j: (0, 0)),
        pl.BlockSpec((1, D), lambda j: (0, 0)),
        pl.BlockSpec((NBLK, D), lambda j: (j, 0)),
    ],
    out_specs=pl.BlockSpec((T, NBLK), lambda j: (0, j)),
    out_shape=jax.ShapeDtypeStruct((T, VOCAB), F32),
    scratch_shapes=[pltpu.VMEM((T, D), BF)],
)


def kernel(ids_t, embed_W, ln_scale, router_W, router_b, W1, b1, W2, b2):
    ids = ids_t.astype(I32)
    h = _embed_gather(embed_W, ids)
    rw = jnp.pad(jnp.transpose(router_W, (0, 2, 1)).astype(BF),
                 ((0, 0), (0, 0), (0, 128 - E)))
    rb = jnp.pad(router_b, ((0, 0), (0, 128 - E)))[:, None, :]
    W1b = W1.astype(BF)
    W2b = W2.astype(BF)
    for hop in range(NH):
        d1, d2, c1, c2, w1, w2, rho = _router(h, rw[hop], rb[hop])
        exp_in = _dispatch(h, d1.reshape(T), d2.reshape(T))
        eo = _ffn(exp_in.reshape(E, C, D), W1b[hop], b1[hop].reshape(E, 1, F),
                  W2b[hop], b2[hop].reshape(E, 1, D))
        h = _combine(h, eo.reshape(EC, D), c1.reshape(T), c2.reshape(T),
                     w1.reshape(T), w2.reshape(T), rho.reshape(T))
    return _final(h, ln_scale[None], embed_W.astype(BF))
